# baseline (speedup 1.0000x reference)
import functools

import jax
import jax.numpy as jnp
import numpy as np
from jax import lax
from jax.experimental import pallas as pl
from jax.experimental.pallas import tpu as pltpu
from jax.experimental.pallas import tpu_sc as plsc

N_FIELDS = 26
FIELD_DIM = 100000
EMBED = 32
N_FLOAT = 13
BATCH = 16384

NUM_CORES = 2
NUM_SUBCORES = 16
NW = NUM_CORES * NUM_SUBCORES
B_W = BATCH // NW
ROWS_W = B_W * N_FIELDS
PAT = 208
GCHUNK = 128
GROUP = 8
WCHUNK = GCHUNK * GROUP
NWRITE = ROWS_W // WCHUNK
TROWS = N_FIELDS * FIELD_DIM

NGRP = TROWS // 8
CH = 52
NCH = NGRP // CH


def _sc_passthru(table3):
    mesh = plsc.VectorSubcoreMesh(
        core_axis_name="c", subcore_axis_name="s",
        num_cores=NUM_CORES, num_subcores=NUM_SUBCORES)

    @functools.partial(
        pl.kernel,
        mesh=mesh,
        out_type=jax.ShapeDtypeStruct((NGRP, 8, EMBED), jnp.float32),
        scratch_types=[
            pltpu.VMEM((CH, 8, EMBED), jnp.float32),
            pltpu.VMEM((CH, 8, EMBED), jnp.float32),
            pltpu.SemaphoreType.DMA,
            pltpu.SemaphoreType.DMA,
        ],
        compiler_params=pltpu.CompilerParams(needs_layout_passes=False),
    )
    def k(tab_hbm, out_hbm, va, vb, sa, sb):
        wid = lax.axis_index("s") * NUM_CORES + lax.axis_index("c")

        def fire(c, buf, sem):
            @pl.when(c < NCH)
            def _():
                pltpu.make_async_copy(
                    tab_hbm.at[pl.ds(c * CH, CH)], buf, sem).start()

        def drain(c, buf, sem):
            @pl.when(c < NCH)
            def _():
                pltpu.make_async_copy(
                    tab_hbm.at[pl.ds(c * CH, CH)], buf, sem).wait()

        def emit(c, buf):
            @pl.when(c < NCH)
            def _():
                pltpu.sync_copy(buf, out_hbm.at[pl.ds(c * CH, CH)])

        fire(wid, va, sa)
        def outer(i, carry):
            cA = wid + i * 64
            cB = cA + 32
            fire(cB, vb, sb)
            drain(cA, va, sa)
            emit(cA, va)
            fire(cA + 64, va, sa)
            drain(cB, vb, sb)
            emit(cB, vb)
            return carry
        lax.fori_loop(0, (NCH + 63) // 64, outer, 0)

    return k(table3)


def _sc_gather(tok_flat, pat, packed2):
    mesh = plsc.VectorSubcoreMesh(
        core_axis_name="c", subcore_axis_name="s",
        num_cores=NUM_CORES, num_subcores=NUM_SUBCORES)

    @functools.partial(
        pl.kernel,
        mesh=mesh,
        out_type=jax.ShapeDtypeStruct((BATCH * N_FIELDS, EMBED), jnp.float32),
        scratch_types=[
            pltpu.VMEM((ROWS_W,), jnp.int32),
            pltpu.VMEM((PAT,), jnp.int32),
            pltpu.VMEM((WCHUNK, EMBED), jnp.float32),
            pltpu.VMEM((WCHUNK, EMBED), jnp.float32),
            pltpu.SemaphoreType.DMA,
            pltpu.SemaphoreType.DMA,
        ],
        compiler_params=pltpu.CompilerParams(use_tc_tiling_on_sc=False),
    )
    def k(tok_hbm, pat_hbm, table_hbm, out_hbm, idx_v, pat_v, w0, w1, s0, s1):
        wid = lax.axis_index("s") * NUM_CORES + lax.axis_index("c")
        base = wid * ROWS_W
        pltpu.sync_copy(tok_hbm.at[pl.ds(base, ROWS_W)], idx_v)
        pltpu.sync_copy(pat_hbm, pat_v)

        def add_body(g, carry):
            for v in range(PAT // 16):
                sl = pl.ds(g * PAT + v * 16, 16)
                idx_v[sl] = idx_v[sl] + pat_v[pl.ds(v * 16, 16)]
            return carry
        lax.fori_loop(0, ROWS_W // PAT, add_body, 0)

        bufs = (w0, w1)
        sems = (s0, s1)

        def fire(c):
            p = c % 2
            hs = []
            for sub in range(GROUP):
                r0 = c * WCHUNK + sub * GCHUNK
                hs.append(pltpu.async_copy(
                    table_hbm.at[idx_v.at[pl.ds(r0, GCHUNK)]],
                    bufs[p].at[pl.ds(sub * GCHUNK, GCHUNK)],
                    sems[p]))
            return hs

        pending = {0: fire(0)}
        for c in range(NWRITE):
            if c + 1 < NWRITE:
                pending[c + 1] = fire(c + 1)
            for h in pending.pop(c):
                h.wait()
            pltpu.sync_copy(bufs[c % 2],
                            out_hbm.at[pl.ds(base + c * WCHUNK, WCHUNK)])

    return k(tok_flat, pat, packed2)


def _tc_dense(ff, W, b2d):
    def body(ff_ref, w_ref, b_ref, o_ref):
        o_ref[...] = jnp.dot(ff_ref[...], w_ref[...],
                             preferred_element_type=jnp.float32) + b_ref[...]

    return pl.pallas_call(
        body,
        out_shape=jax.ShapeDtypeStruct((BATCH, EMBED), jnp.float32),
    )(ff, W, b2d)


def kernel(token_fields, float_fields, table, W_float, b_float):
    tok_flat = token_fields.astype(jnp.int32).reshape(-1)
    pat = jnp.asarray(
        np.tile(np.arange(N_FIELDS, dtype=np.int32) * FIELD_DIM,
                PAT // N_FIELDS))
    packed = _sc_passthru(table.reshape(NGRP, 8, EMBED))
    packed2 = packed.reshape(TROWS, EMBED)
    sparse = _sc_gather(tok_flat, pat, packed2)
    dense = _tc_dense(float_fields, W_float, b_float.reshape(1, EMBED))
    return jnp.concatenate(
        [sparse.reshape(BATCH, N_FIELDS, EMBED), dense[:, None, :]], axis=1)


# speedup vs baseline: 1.5017x; 1.5017x over previous
import functools

import jax
import jax.numpy as jnp
import numpy as np
from jax import lax
from jax.experimental import pallas as pl
from jax.experimental.pallas import tpu as pltpu
from jax.experimental.pallas import tpu_sc as plsc

N_FIELDS = 26
FIELD_DIM = 100000
EMBED = 32
N_FLOAT = 13
BATCH = 16384

NUM_CORES = 2
NUM_SUBCORES = 16
NW = NUM_CORES * NUM_SUBCORES
B_W = BATCH // NW
ROWS_W = B_W * N_FIELDS
PAT = 208
GCHUNK = 128
GROUP = 8
WCHUNK = GCHUNK * GROUP
NWRITE = ROWS_W // WCHUNK
TROWS = N_FIELDS * FIELD_DIM

NGRP = TROWS // 8
CH = 52
NCH = NGRP // CH


def _sc_passthru(table3):
    mesh = plsc.VectorSubcoreMesh(
        core_axis_name="c", subcore_axis_name="s",
        num_cores=NUM_CORES, num_subcores=NUM_SUBCORES)

    @functools.partial(
        pl.kernel,
        mesh=mesh,
        out_type=jax.ShapeDtypeStruct((TROWS // 4, 128), jnp.float32),
        scratch_types=[
            pltpu.VMEM((CH * 8, EMBED), jnp.float32),
            pltpu.VMEM((CH * 8, EMBED), jnp.float32),
            pltpu.VMEM((CH * 2, 128), jnp.float32),
            pltpu.SemaphoreType.DMA,
            pltpu.SemaphoreType.DMA,
        ],
    )
    def k(tab_hbm, out_hbm, va, vb, vp, sa, sb):
        wid = lax.axis_index("s") * NUM_CORES + lax.axis_index("c")

        def fire(c, buf, sem):
            @pl.when(c < NCH)
            def _():
                pltpu.make_async_copy(
                    tab_hbm.at[pl.ds(c * CH * 8, CH * 8)], buf, sem).start()

        def drain(c, buf, sem):
            @pl.when(c < NCH)
            def _():
                pltpu.make_async_copy(
                    tab_hbm.at[pl.ds(c * CH * 8, CH * 8)], buf, sem).wait()

        def emit(c, buf):
            @pl.when(c < NCH)
            def _():
                for r in range(CH * 8):
                    p = r // 4
                    l = (r % 4) * EMBED
                    vp[p, pl.ds(l, 16)] = buf[r, pl.ds(0, 16)]
                    vp[p, pl.ds(l + 16, 16)] = buf[r, pl.ds(16, 16)]
                pltpu.sync_copy(vp, out_hbm.at[pl.ds(c * CH * 2, CH * 2)])

        fire(wid, va, sa)
        def outer(i, carry):
            cA = wid + i * 64
            cB = cA + 32
            fire(cB, vb, sb)
            drain(cA, va, sa)
            emit(cA, va)
            fire(cA + 64, va, sa)
            drain(cB, vb, sb)
            emit(cB, vb)
            return carry
        lax.fori_loop(0, (NCH + 63) // 64, outer, 0)

    return k(table3)


def _sc_gather(tok_flat, pat, packed2):
    mesh = plsc.VectorSubcoreMesh(
        core_axis_name="c", subcore_axis_name="s",
        num_cores=NUM_CORES, num_subcores=NUM_SUBCORES)

    @functools.partial(
        pl.kernel,
        mesh=mesh,
        out_type=jax.ShapeDtypeStruct((BATCH * N_FIELDS, EMBED), jnp.float32),
        scratch_types=[
            pltpu.VMEM((ROWS_W,), jnp.int32),
            pltpu.VMEM((PAT,), jnp.int32),
            pltpu.VMEM((WCHUNK, EMBED), jnp.float32),
            pltpu.VMEM((WCHUNK, EMBED), jnp.float32),
            pltpu.SemaphoreType.DMA,
            pltpu.SemaphoreType.DMA,
        ],
        compiler_params=pltpu.CompilerParams(use_tc_tiling_on_sc=False),
    )
    def k(tok_hbm, pat_hbm, table_hbm, out_hbm, idx_v, pat_v, w0, w1, s0, s1):
        wid = lax.axis_index("s") * NUM_CORES + lax.axis_index("c")
        base = wid * ROWS_W
        pltpu.sync_copy(tok_hbm.at[pl.ds(base, ROWS_W)], idx_v)
        pltpu.sync_copy(pat_hbm, pat_v)

        def add_body(g, carry):
            for v in range(PAT // 16):
                sl = pl.ds(g * PAT + v * 16, 16)
                idx_v[sl] = idx_v[sl] + pat_v[pl.ds(v * 16, 16)]
            return carry
        lax.fori_loop(0, ROWS_W // PAT, add_body, 0)

        bufs = (w0, w1)
        sems = (s0, s1)

        def fire(c):
            p = c % 2
            hs = []
            for sub in range(GROUP):
                r0 = c * WCHUNK + sub * GCHUNK
                hs.append(pltpu.async_copy(
                    table_hbm.at[idx_v.at[pl.ds(r0, GCHUNK)]],
                    bufs[p].at[pl.ds(sub * GCHUNK, GCHUNK)],
                    sems[p]))
            return hs

        pending = {0: fire(0)}
        for c in range(NWRITE):
            if c + 1 < NWRITE:
                pending[c + 1] = fire(c + 1)
            for h in pending.pop(c):
                h.wait()
            pltpu.sync_copy(bufs[c % 2],
                            out_hbm.at[pl.ds(base + c * WCHUNK, WCHUNK)])

    return k(tok_flat, pat, packed2)


def _tc_dense(ff, W, b2d):
    def body(ff_ref, w_ref, b_ref, o_ref):
        o_ref[...] = jnp.dot(ff_ref[...], w_ref[...],
                             preferred_element_type=jnp.float32) + b_ref[...]

    return pl.pallas_call(
        body,
        out_shape=jax.ShapeDtypeStruct((BATCH, EMBED), jnp.float32),
    )(ff, W, b2d)


def kernel(token_fields, float_fields, table, W_float, b_float):
    tok_flat = token_fields.astype(jnp.int32).reshape(-1)
    pat = jnp.asarray(
        np.tile(np.arange(N_FIELDS, dtype=np.int32) * FIELD_DIM,
                PAT // N_FIELDS))
    packed = _sc_passthru(table)
    packed2 = packed.reshape(TROWS, EMBED)
    sparse = _sc_gather(tok_flat, pat, packed2)
    dense = _tc_dense(float_fields, W_float, b_float.reshape(1, EMBED))
    return jnp.concatenate(
        [sparse.reshape(BATCH, N_FIELDS, EMBED), dense[:, None, :]], axis=1)


# speedup vs baseline: 1.8868x; 1.2564x over previous
import functools

import jax
import jax.numpy as jnp
import numpy as np
from jax import lax
from jax.experimental import pallas as pl
from jax.experimental.pallas import tpu as pltpu
from jax.experimental.pallas import tpu_sc as plsc

N_FIELDS = 26
FIELD_DIM = 100000
EMBED = 32
N_FLOAT = 13
BATCH = 16384

NUM_CORES = 2
NUM_SUBCORES = 16
NW = NUM_CORES * NUM_SUBCORES
B_W = BATCH // NW
ROWS_W = B_W * N_FIELDS
PAT = 208
GCHUNK = 128
GROUP = 8
WCHUNK = GCHUNK * GROUP
NWRITE = ROWS_W // WCHUNK
TROWS = N_FIELDS * FIELD_DIM

NGRP = TROWS // 8
CH = 52
NCH = NGRP // CH


def _sc_passthru(table3):
    mesh = plsc.VectorSubcoreMesh(
        core_axis_name="c", subcore_axis_name="s",
        num_cores=NUM_CORES, num_subcores=NUM_SUBCORES)

    @functools.partial(
        pl.kernel,
        mesh=mesh,
        out_type=jax.ShapeDtypeStruct((TROWS // 4, 128), jnp.float32),
        scratch_types=[
            pltpu.VMEM((CH, 8, EMBED), jnp.float32),
            pltpu.VMEM((CH, 8, EMBED), jnp.float32),
            pltpu.VMEM((CH * 2, 128), jnp.float32),
            pltpu.SemaphoreType.DMA,
            pltpu.SemaphoreType.DMA,
        ],
        compiler_params=pltpu.CompilerParams(needs_layout_passes=False),
    )
    def k(tab_hbm, out_hbm, va, vb, vp, sa, sb):
        wid = lax.axis_index("s") * NUM_CORES + lax.axis_index("c")

        def fire(c, buf, sem):
            @pl.when(c < NCH)
            def _():
                pltpu.make_async_copy(
                    tab_hbm.at[pl.ds(c * CH, CH)], buf, sem).start()

        def drain(c, buf, sem):
            @pl.when(c < NCH)
            def _():
                pltpu.make_async_copy(
                    tab_hbm.at[pl.ds(c * CH, CH)], buf, sem).wait()

        def emit(c, buf):
            @pl.when(c < NCH)
            def _():
                for r in range(CH * 8):
                    p = r // 4
                    l = (r % 4) * EMBED
                    vp[p, pl.ds(l, 16)] = buf[r // 8, r % 8, pl.ds(0, 16)]
                    vp[p, pl.ds(l + 16, 16)] = buf[r // 8, r % 8,
                                                   pl.ds(16, 16)]
                pltpu.sync_copy(vp, out_hbm.at[pl.ds(c * CH * 2, CH * 2)])

        fire(wid, va, sa)
        def outer(i, carry):
            cA = wid + i * 64
            cB = cA + 32
            fire(cB, vb, sb)
            drain(cA, va, sa)
            emit(cA, va)
            fire(cA + 64, va, sa)
            drain(cB, vb, sb)
            emit(cB, vb)
            return carry
        lax.fori_loop(0, (NCH + 63) // 64, outer, 0)

    return k(table3)


def _sc_gather(tok_flat, pat, packed2):
    mesh = plsc.VectorSubcoreMesh(
        core_axis_name="c", subcore_axis_name="s",
        num_cores=NUM_CORES, num_subcores=NUM_SUBCORES)

    @functools.partial(
        pl.kernel,
        mesh=mesh,
        out_type=jax.ShapeDtypeStruct((BATCH * N_FIELDS, EMBED), jnp.float32),
        scratch_types=[
            pltpu.VMEM((ROWS_W,), jnp.int32),
            pltpu.VMEM((PAT,), jnp.int32),
            pltpu.VMEM((WCHUNK, EMBED), jnp.float32),
            pltpu.VMEM((WCHUNK, EMBED), jnp.float32),
            pltpu.SemaphoreType.DMA,
            pltpu.SemaphoreType.DMA,
        ],
        compiler_params=pltpu.CompilerParams(use_tc_tiling_on_sc=False),
    )
    def k(tok_hbm, pat_hbm, table_hbm, out_hbm, idx_v, pat_v, w0, w1, s0, s1):
        wid = lax.axis_index("s") * NUM_CORES + lax.axis_index("c")
        base = wid * ROWS_W
        pltpu.sync_copy(tok_hbm.at[pl.ds(base, ROWS_W)], idx_v)
        pltpu.sync_copy(pat_hbm, pat_v)

        def add_body(g, carry):
            for v in range(PAT // 16):
                sl = pl.ds(g * PAT + v * 16, 16)
                idx_v[sl] = idx_v[sl] + pat_v[pl.ds(v * 16, 16)]
            return carry
        lax.fori_loop(0, ROWS_W // PAT, add_body, 0)

        bufs = (w0, w1)
        sems = (s0, s1)

        def fire(c):
            p = c % 2
            hs = []
            for sub in range(GROUP):
                r0 = c * WCHUNK + sub * GCHUNK
                hs.append(pltpu.async_copy(
                    table_hbm.at[idx_v.at[pl.ds(r0, GCHUNK)]],
                    bufs[p].at[pl.ds(sub * GCHUNK, GCHUNK)],
                    sems[p]))
            return hs

        pending = {0: fire(0)}
        for c in range(NWRITE):
            if c + 1 < NWRITE:
                pending[c + 1] = fire(c + 1)
            for h in pending.pop(c):
                h.wait()
            pltpu.sync_copy(bufs[c % 2],
                            out_hbm.at[pl.ds(base + c * WCHUNK, WCHUNK)])

    return k(tok_flat, pat, packed2)


def _tc_dense(ff, W, b2d):
    def body(ff_ref, w_ref, b_ref, o_ref):
        o_ref[...] = jnp.dot(ff_ref[...], w_ref[...],
                             preferred_element_type=jnp.float32) + b_ref[...]

    return pl.pallas_call(
        body,
        out_shape=jax.ShapeDtypeStruct((BATCH, EMBED), jnp.float32),
    )(ff, W, b2d)


def kernel(token_fields, float_fields, table, W_float, b_float):
    tok_flat = token_fields.astype(jnp.int32).reshape(-1)
    pat = jnp.asarray(
        np.tile(np.arange(N_FIELDS, dtype=np.int32) * FIELD_DIM,
                PAT // N_FIELDS))
    packed = _sc_passthru(table.reshape(NGRP, 8, EMBED))
    packed2 = packed.reshape(TROWS, EMBED)
    sparse = _sc_gather(tok_flat, pat, packed2)
    dense = _tc_dense(float_fields, W_float, b_float.reshape(1, EMBED))
    return jnp.concatenate(
        [sparse.reshape(BATCH, N_FIELDS, EMBED), dense[:, None, :]], axis=1)


# speedup vs baseline: 1.9093x; 1.0119x over previous
import functools

import jax
import jax.numpy as jnp
import numpy as np
from jax import lax
from jax.experimental import pallas as pl
from jax.experimental.pallas import tpu as pltpu
from jax.experimental.pallas import tpu_sc as plsc

N_FIELDS = 26
FIELD_DIM = 100000
EMBED = 32
N_FLOAT = 13
BATCH = 16384

NUM_CORES = 2
NUM_SUBCORES = 16
NW = NUM_CORES * NUM_SUBCORES
B_W = BATCH // NW
ROWS_W = B_W * N_FIELDS
PAT = 208
GCHUNK = 128
GROUP = 8
WCHUNK = GCHUNK * GROUP
NWRITE = ROWS_W // WCHUNK
TROWS = N_FIELDS * FIELD_DIM

NGRP = TROWS // 8
CH = 40
NCH = NGRP // CH


def _sc_passthru(table3):
    mesh = plsc.VectorSubcoreMesh(
        core_axis_name="c", subcore_axis_name="s",
        num_cores=NUM_CORES, num_subcores=NUM_SUBCORES)

    @functools.partial(
        pl.kernel,
        mesh=mesh,
        out_type=jax.ShapeDtypeStruct((TROWS // 4, 128), jnp.float32),
        scratch_types=[
            pltpu.VMEM((CH, 8, EMBED), jnp.float32),
            pltpu.VMEM((CH, 8, EMBED), jnp.float32),
            pltpu.VMEM((CH * 2, 128), jnp.float32),
            pltpu.VMEM((CH * 2, 128), jnp.float32),
            pltpu.SemaphoreType.DMA,
            pltpu.SemaphoreType.DMA,
            pltpu.SemaphoreType.DMA,
            pltpu.SemaphoreType.DMA,
        ],
        compiler_params=pltpu.CompilerParams(needs_layout_passes=False),
    )
    def k(tab_hbm, out_hbm, va, vb, vpa, vpb, sa, sb, soa, sob):
        wid = lax.axis_index("s") * NUM_CORES + lax.axis_index("c")

        def fire(c, buf, sem):
            @pl.when(c < NCH)
            def _():
                pltpu.make_async_copy(
                    tab_hbm.at[pl.ds(c * CH, CH)], buf, sem).start()

        def drain(c, buf, sem):
            @pl.when(c < NCH)
            def _():
                pltpu.make_async_copy(
                    tab_hbm.at[pl.ds(c * CH, CH)], buf, sem).wait()

        def emit(c, buf, vp, osem):
            @pl.when(c < NCH)
            def _():
                for r in range(CH * 8):
                    p = r // 4
                    l = (r % 4) * EMBED
                    vp[p, pl.ds(l, 16)] = buf[r // 8, r % 8, pl.ds(0, 16)]
                    vp[p, pl.ds(l + 16, 16)] = buf[r // 8, r % 8,
                                                   pl.ds(16, 16)]
                pltpu.make_async_copy(
                    vp, out_hbm.at[pl.ds(c * CH * 2, CH * 2)], osem).start()

        def emit_wait(c, vp, osem):
            @pl.when(jnp.logical_and(c >= 0, c < NCH))
            def _():
                pltpu.make_async_copy(
                    vp, out_hbm.at[pl.ds(c * CH * 2, CH * 2)], osem).wait()

        fire(wid, va, sa)
        def outer(i, carry):
            cA = wid + i * 64
            cB = cA + 32
            fire(cB, vb, sb)
            drain(cA, va, sa)
            emit_wait(cA - 64, vpa, soa)
            emit(cA, va, vpa, soa)
            fire(cA + 64, va, sa)
            drain(cB, vb, sb)
            emit_wait(cB - 64, vpb, sob)
            emit(cB, vb, vpb, sob)
            return carry
        NITER = (NCH + 63) // 64
        lax.fori_loop(0, NITER, outer, 0)
        emit_wait(wid + (NITER - 1) * 64, vpa, soa)
        emit_wait(wid + (NITER - 1) * 64 + 32, vpb, sob)

    return k(table3)


def _sc_gather(tok_flat, pat, packed2):
    mesh = plsc.VectorSubcoreMesh(
        core_axis_name="c", subcore_axis_name="s",
        num_cores=NUM_CORES, num_subcores=NUM_SUBCORES)

    @functools.partial(
        pl.kernel,
        mesh=mesh,
        out_type=jax.ShapeDtypeStruct((BATCH * N_FIELDS, EMBED), jnp.float32),
        scratch_types=[
            pltpu.VMEM((ROWS_W,), jnp.int32),
            pltpu.VMEM((PAT,), jnp.int32),
            pltpu.VMEM((WCHUNK, EMBED), jnp.float32),
            pltpu.VMEM((WCHUNK, EMBED), jnp.float32),
            pltpu.SemaphoreType.DMA,
            pltpu.SemaphoreType.DMA,
        ],
        compiler_params=pltpu.CompilerParams(use_tc_tiling_on_sc=False),
    )
    def k(tok_hbm, pat_hbm, table_hbm, out_hbm, idx_v, pat_v, w0, w1, s0, s1):
        wid = lax.axis_index("s") * NUM_CORES + lax.axis_index("c")
        base = wid * ROWS_W
        pltpu.sync_copy(tok_hbm.at[pl.ds(base, ROWS_W)], idx_v)
        pltpu.sync_copy(pat_hbm, pat_v)

        def add_body(g, carry):
            for v in range(PAT // 16):
                sl = pl.ds(g * PAT + v * 16, 16)
                idx_v[sl] = idx_v[sl] + pat_v[pl.ds(v * 16, 16)]
            return carry
        lax.fori_loop(0, ROWS_W // PAT, add_body, 0)

        bufs = (w0, w1)
        sems = (s0, s1)

        def fire(c):
            p = c % 2
            hs = []
            for sub in range(GROUP):
                r0 = c * WCHUNK + sub * GCHUNK
                hs.append(pltpu.async_copy(
                    table_hbm.at[idx_v.at[pl.ds(r0, GCHUNK)]],
                    bufs[p].at[pl.ds(sub * GCHUNK, GCHUNK)],
                    sems[p]))
            return hs

        pending = {0: fire(0)}
        for c in range(NWRITE):
            if c + 1 < NWRITE:
                pending[c + 1] = fire(c + 1)
            for h in pending.pop(c):
                h.wait()
            pltpu.sync_copy(bufs[c % 2],
                            out_hbm.at[pl.ds(base + c * WCHUNK, WCHUNK)])

    return k(tok_flat, pat, packed2)


def _tc_dense(ff, W, b2d):
    def body(ff_ref, w_ref, b_ref, o_ref):
        o_ref[...] = jnp.dot(ff_ref[...], w_ref[...],
                             preferred_element_type=jnp.float32) + b_ref[...]

    return pl.pallas_call(
        body,
        out_shape=jax.ShapeDtypeStruct((BATCH, EMBED), jnp.float32),
    )(ff, W, b2d)


def kernel(token_fields, float_fields, table, W_float, b_float):
    tok_flat = token_fields.astype(jnp.int32).reshape(-1)
    pat = jnp.asarray(
        np.tile(np.arange(N_FIELDS, dtype=np.int32) * FIELD_DIM,
                PAT // N_FIELDS))
    packed = _sc_passthru(table.reshape(NGRP, 8, EMBED))
    packed2 = packed.reshape(TROWS, EMBED)
    sparse = _sc_gather(tok_flat, pat, packed2)
    dense = _tc_dense(float_fields, W_float, b_float.reshape(1, EMBED))
    return jnp.concatenate(
        [sparse.reshape(BATCH, N_FIELDS, EMBED), dense[:, None, :]], axis=1)
